# baseline (speedup 1.0000x reference)
import functools

import jax
import jax.numpy as jnp
from jax import lax
from jax.experimental import pallas as pl
from jax.experimental.pallas import tpu as pltpu
from jax.experimental.pallas import tpu_sc as plsc

_NC = 2
_NS = 16
_NW = _NC * _NS


def _make_gather(B, V, D, C):
    assert B % (_NW * C) == 0
    b_per_w = B // _NW
    n_chunks = b_per_w // C
    mesh = plsc.VectorSubcoreMesh(core_axis_name="c", subcore_axis_name="s")

    @functools.partial(
        pl.kernel,
        mesh=mesh,
        out_type=jax.ShapeDtypeStruct((B, D), jnp.float32),
        scratch_types=[
            pltpu.VMEM((C,), jnp.int32),
            pltpu.VMEM((C, D), jnp.float32),
            pltpu.SemaphoreType.DMA,
        ],
        compiler_params=pltpu.CompilerParams(use_tc_tiling_on_sc=False),
    )
    def gather_kernel(idx_hbm, table_hbm, out_hbm, idx_v, rows_v, sem):
        wid = lax.axis_index("s") * _NC + lax.axis_index("c")
        base = wid * b_per_w

        def body(j, carry):
            off = base + j * C
            pltpu.sync_copy(idx_hbm.at[pl.ds(off, C)], idx_v)
            pltpu.async_copy(table_hbm.at[idx_v], rows_v, sem).wait()
            pltpu.sync_copy(rows_v, out_hbm.at[pl.ds(off, C)])
            return carry

        lax.fori_loop(0, n_chunks, body, 0)

    return gather_kernel


def kernel(input, table):
    orig_shape = input.shape
    idx = input.reshape(-1).astype(jnp.int32)
    B = idx.shape[0]
    V, D = table.shape
    out = _make_gather(B, V, D, 512)(idx, table)
    return out.reshape(*orig_shape, D)


# speedup vs baseline: 1.0238x; 1.0238x over previous
import functools

import jax
import jax.numpy as jnp
from jax import lax
from jax.experimental import pallas as pl
from jax.experimental.pallas import tpu as pltpu
from jax.experimental.pallas import tpu_sc as plsc

_NC = 2
_NS = 16
_NW = _NC * _NS


def _make_gather(B, V, D, C, NB):
    assert B % (_NW * C * NB) == 0
    b_per_w = B // _NW
    n_chunks = b_per_w // C
    n_groups = n_chunks // NB
    mesh = plsc.VectorSubcoreMesh(core_axis_name="c", subcore_axis_name="s")

    scratch = (
        [pltpu.VMEM((b_per_w,), jnp.int32)]
        + [pltpu.VMEM((C, D), jnp.float32) for _ in range(NB)]
        + [pltpu.SemaphoreType.DMA for _ in range(2 * NB)]
    )

    @functools.partial(
        pl.kernel,
        mesh=mesh,
        out_type=jax.ShapeDtypeStruct((B, D), jnp.float32),
        scratch_types=scratch,
        compiler_params=pltpu.CompilerParams(use_tc_tiling_on_sc=False),
    )
    def gather_kernel(idx_hbm, table_hbm, out_hbm, idx_v, *rest):
        bufs = rest[:NB]
        gsems = rest[NB:2 * NB]
        osems = rest[2 * NB:]
        wid = lax.axis_index("s") * _NC + lax.axis_index("c")
        base = wid * b_per_w
        pltpu.sync_copy(idx_hbm.at[pl.ds(base, b_per_w)], idx_v)

        def gather(j, b):
            pltpu.async_copy(
                table_hbm.at[idx_v.at[pl.ds(j * C, C)]], bufs[b], gsems[b])

        def wait_gather(j, b):
            pltpu.make_async_copy(
                table_hbm.at[idx_v.at[pl.ds(j * C, C)]], bufs[b],
                gsems[b]).wait()

        def write(j, b):
            pltpu.async_copy(
                bufs[b], out_hbm.at[pl.ds(base + j * C, C)], osems[b])

        def wait_write(j, b):
            pltpu.make_async_copy(
                bufs[b], out_hbm.at[pl.ds(base + j * C, C)], osems[b]).wait()

        for b in range(NB):
            gather(b, b)

        def body(g, carry):
            for b in range(NB):
                wait_gather(g * NB + b, b)
                write(g * NB + b, b)

            @pl.when(g + 1 < n_groups)
            def _():
                for b in range(NB):
                    wait_write(g * NB + b, b)
                    gather((g + 1) * NB + b, b)

            return carry

        lax.fori_loop(0, n_groups, body, 0)
        for b in range(NB):
            wait_write((n_groups - 1) * NB + b, b)

    return gather_kernel


def kernel(input, table):
    orig_shape = input.shape
    idx = input.reshape(-1).astype(jnp.int32)
    B = idx.shape[0]
    V, D = table.shape
    out = _make_gather(B, V, D, 416, 4)(idx, table)
    return out.reshape(*orig_shape, D)
